# baseline (speedup 1.0000x reference)
import functools

import jax
import jax.numpy as jnp
from jax import lax
from jax.experimental import pallas as pl
from jax.experimental.pallas import tpu as pltpu
from jax.experimental.pallas import tpu_sc as plsc

VOCAB = 100000
EMB = 128
BATCH = 16384

NC = 2
NS = 16
L = 16
NW = NC * NS
BPW = BATCH // NW
CHUNK = 128
NCHUNK = BPW // CHUNK
NACC = 4


def _group_scatter(t_buf, c_buf, w_t, w_c, lane16, tr, r0):
    for rr in range(L):
        r = r0 + rr
        accs = [jnp.zeros((L,), jnp.float32) for _ in range(NACC)]
        for k in range(EMB // L):
            vt = t_buf[r, pl.ds(k * L, L)]
            accs[k % NACC] = accs[k % NACC] + vt * w_t[k]
        for k in range(EMB // L):
            vc = c_buf[r, pl.ds(k * L, L)]
            accs[(k + 2) % NACC] = accs[(k + 2) % NACC] + vc * w_c[k]
        part = (accs[0] + accs[1]) + (accs[2] + accs[3])
        plsc.store_scatter(tr, [lane16 + rr], part)


def _consume(tr, b_s, out_v, off):
    sums = [tr[pl.ds(l * L, L)] for l in range(0, L, NACC)]
    for l in range(L):
        if l % NACC:
            sums[l // NACC] = sums[l // NACC] + tr[pl.ds(l * L, L)]
    x = (sums[0] + sums[1]) + (sums[2] + sums[3]) + b_s
    out_v[pl.ds(off, L)] = 1.0 / (1.0 + jnp.exp(-x))


def _chunk_compute(t_buf, c_buf, w_t, w_c, b_s, lane, tr_a, tr_b, out_v,
                   c_base):
    lane16 = lane * L

    def hbody(h, carry):
        g0 = 2 * h
        _group_scatter(t_buf, c_buf, w_t, w_c, lane16, tr_a, g0 * L)

        @pl.when(h > 0)
        def _():
            _consume(tr_b, b_s, out_v, c_base + (g0 - 1) * L)

        _group_scatter(t_buf, c_buf, w_t, w_c, lane16, tr_b, (g0 + 1) * L)
        _consume(tr_a, b_s, out_v, c_base + g0 * L)
        return carry

    lax.fori_loop(0, CHUNK // (2 * L), hbody, 0, unroll=False)
    _consume(tr_b, b_s, out_v, c_base + CHUNK - L)


def _body(offset, in_hbm, tt_hbm, ct_hbm, w_hbm, b_hbm, out_hbm,
          iv_v, t_idx_v, c_idx_v, w_v, b16_v, t_buf, c_buf, tr_a, tr_b,
          out_v, sem_t0, sem_t1, sem_c0, sem_c1):
    wid = lax.axis_index("s")
    lbase = wid * BPW
    base = offset + lbase
    lane = lax.iota(jnp.int32, L)

    pltpu.sync_copy(w_hbm.at[0], w_v)
    pltpu.sync_copy(b_hbm, b16_v.at[pl.ds(0, 1)])
    pltpu.sync_copy(in_hbm.at[pl.ds(2 * base, 2 * BPW)], iv_v)

    def dbody(g, rows2):
        t_idx_v[pl.ds(g * L, L)] = plsc.load_gather(iv_v, [rows2])
        c_idx_v[pl.ds(g * L, L)] = plsc.load_gather(iv_v, [rows2 + 1])
        return rows2 + 2 * L

    lax.fori_loop(0, BPW // L, dbody, lane * 2, unroll=False)

    sems = [(sem_t0, sem_c0), (sem_t1, sem_c1)]

    def start(c):
        s = c % 2
        ht = pltpu.async_copy(tt_hbm.at[t_idx_v.at[pl.ds(c * CHUNK, CHUNK)]],
                              t_buf.at[s], sems[s][0])
        hc = pltpu.async_copy(ct_hbm.at[c_idx_v.at[pl.ds(c * CHUNK, CHUNK)]],
                              c_buf.at[s], sems[s][1])
        return ht, hc

    pending = {0: start(0)}
    b_s = b16_v[pl.ds(0, L)][0]
    w_t = [w_v[pl.ds(k * L, L)] for k in range(EMB // L)]
    w_c = [w_v[pl.ds(EMB + k * L, L)] for k in range(EMB // L)]

    for c in range(NCHUNK):
        if c + 1 < NCHUNK:
            pending[c + 1] = start(c + 1)
        ht, hc = pending.pop(c)
        ht.wait()
        hc.wait()
        s = c % 2
        _chunk_compute(t_buf.at[s], c_buf.at[s], w_t, w_c, b_s, lane,
                       tr_a, tr_b, out_v, c * CHUNK)

    pltpu.sync_copy(out_v, out_hbm.at[pl.ds(lbase, BPW)])


def _make_half(offset):
    mesh = plsc.VectorSubcoreMesh(core_axis_name="c", subcore_axis_name="s",
                                  num_cores=1)
    return pl.kernel(
        functools.partial(_body, offset),
        mesh=mesh,
        compiler_params=pltpu.CompilerParams(needs_layout_passes=False),
        out_type=jax.ShapeDtypeStruct((BATCH // 2,), jnp.float32),
        scratch_types=[
            pltpu.VMEM((2 * BPW,), jnp.int32),
            pltpu.VMEM((BPW,), jnp.int32),
            pltpu.VMEM((BPW,), jnp.int32),
            pltpu.VMEM((2 * EMB,), jnp.float32),
            pltpu.VMEM((L,), jnp.float32),
            pltpu.VMEM((2, CHUNK, EMB), jnp.float32),
            pltpu.VMEM((2, CHUNK, EMB), jnp.float32),
            pltpu.VMEM((L * L,), jnp.float32),
            pltpu.VMEM((L * L,), jnp.float32),
            pltpu.VMEM((BPW,), jnp.float32),
            pltpu.SemaphoreType.DMA,
            pltpu.SemaphoreType.DMA,
            pltpu.SemaphoreType.DMA,
            pltpu.SemaphoreType.DMA,
        ],
    )


_half0 = _make_half(0)
_half1 = _make_half(BATCH // 2)


@jax.jit
def _run(in_flat, target_table, context_table, W, b):
    o0 = _half0(in_flat, target_table, context_table, W, b)
    o1 = _half1(in_flat, target_table, context_table, W, b)
    return jnp.concatenate([o0, o1])


def kernel(inputs, target_table, context_table, W, b):
    in_flat = inputs.astype(jnp.int32).reshape(2 * BATCH)
    out = _run(in_flat, target_table, context_table, W, b)
    return out.reshape(BATCH, 1)


# speedup vs baseline: 1.7776x; 1.7776x over previous
import functools

import jax
import jax.numpy as jnp
from jax import lax
from jax.experimental import pallas as pl
from jax.experimental.pallas import tpu as pltpu
from jax.experimental.pallas import tpu_sc as plsc

VOCAB = 100000
EMB = 128
BATCH = 16384

NC = 2
NS = 16
L = 16
NW = NC * NS
BPW = BATCH // NW
CHUNK = 128
NCHUNK = BPW // CHUNK
NACC = 4


def _group_scatter(t_buf, c_buf, w_t, w_c, lane16, tr, r0):
    for rr in range(L):
        r = r0 + rr
        accs = [jnp.zeros((L,), jnp.float32) for _ in range(NACC)]
        for k in range(EMB // L):
            vt = t_buf[r, pl.ds(k * L, L)]
            accs[k % NACC] = accs[k % NACC] + vt * w_t[k]
        for k in range(EMB // L):
            vc = c_buf[r, pl.ds(k * L, L)]
            accs[(k + 2) % NACC] = accs[(k + 2) % NACC] + vc * w_c[k]
        part = (accs[0] + accs[1]) + (accs[2] + accs[3])
        plsc.store_scatter(tr, [lane16 + rr], part)


def _consume(tr, b_s, out_v, off):
    sums = [tr[pl.ds(l * L, L)] for l in range(0, L, NACC)]
    for l in range(L):
        if l % NACC:
            sums[l // NACC] = sums[l // NACC] + tr[pl.ds(l * L, L)]
    x = (sums[0] + sums[1]) + (sums[2] + sums[3]) + b_s
    out_v[pl.ds(off, L)] = 1.0 / (1.0 + jnp.exp(-x))


def _chunk_compute(t_buf, c_buf, w_t, w_c, b_s, lane, tr_a, tr_b, out_v,
                   c_base):
    lane16 = lane * L

    def hbody(h, carry):
        g0 = 2 * h
        _group_scatter(t_buf, c_buf, w_t, w_c, lane16, tr_a, g0 * L)

        @pl.when(h > 0)
        def _():
            _consume(tr_b, b_s, out_v, c_base + (g0 - 1) * L)

        _group_scatter(t_buf, c_buf, w_t, w_c, lane16, tr_b, (g0 + 1) * L)
        _consume(tr_a, b_s, out_v, c_base + g0 * L)
        return carry

    lax.fori_loop(0, CHUNK // (2 * L), hbody, 0, unroll=False)
    _consume(tr_b, b_s, out_v, c_base + CHUNK - L)


def _body(t_idx_hbm, c_idx_hbm, tt_hbm, ct_hbm, wb_hbm, out_hbm,
          t_idx_v, c_idx_v, wb_v, t_buf, c_buf, tr_a, tr_b,
          out_v, sem_t0, sem_t1, sem_c0, sem_c1):
    wid = lax.axis_index("s") * NC + lax.axis_index("c")
    base = wid * BPW
    lane = lax.iota(jnp.int32, L)

    pltpu.sync_copy(wb_hbm, wb_v)
    pltpu.sync_copy(t_idx_hbm.at[pl.ds(base, BPW)], t_idx_v)
    pltpu.sync_copy(c_idx_hbm.at[pl.ds(base, BPW)], c_idx_v)

    sems = [(sem_t0, sem_c0), (sem_t1, sem_c1)]

    def start(c):
        s = c % 2
        ht = pltpu.async_copy(tt_hbm.at[t_idx_v.at[pl.ds(c * CHUNK, CHUNK)]],
                              t_buf.at[s], sems[s][0])
        hc = pltpu.async_copy(ct_hbm.at[c_idx_v.at[pl.ds(c * CHUNK, CHUNK)]],
                              c_buf.at[s], sems[s][1])
        return ht, hc

    pending = {0: start(0)}
    b_s = wb_v[pl.ds(2 * EMB, L)][0]
    w_t = [wb_v[pl.ds(k * L, L)] for k in range(EMB // L)]
    w_c = [wb_v[pl.ds(EMB + k * L, L)] for k in range(EMB // L)]

    for c in range(NCHUNK):
        if c + 1 < NCHUNK:
            pending[c + 1] = start(c + 1)
        ht, hc = pending.pop(c)
        ht.wait()
        hc.wait()
        s = c % 2
        _chunk_compute(t_buf.at[s], c_buf.at[s], w_t, w_c, b_s, lane,
                       tr_a, tr_b, out_v, c * CHUNK)

    pltpu.sync_copy(out_v, out_hbm.at[pl.ds(base, BPW)])


def _make_kernel():
    mesh = plsc.VectorSubcoreMesh(core_axis_name="c", subcore_axis_name="s")
    return pl.kernel(
        _body,
        mesh=mesh,
        compiler_params=pltpu.CompilerParams(needs_layout_passes=False),
        out_type=jax.ShapeDtypeStruct((BATCH,), jnp.float32),
        scratch_types=[
            pltpu.VMEM((BPW,), jnp.int32),
            pltpu.VMEM((BPW,), jnp.int32),
            pltpu.VMEM((2 * EMB + L,), jnp.float32),
            pltpu.VMEM((2, CHUNK, EMB), jnp.float32),
            pltpu.VMEM((2, CHUNK, EMB), jnp.float32),
            pltpu.VMEM((L * L,), jnp.float32),
            pltpu.VMEM((L * L,), jnp.float32),
            pltpu.VMEM((BPW,), jnp.float32),
            pltpu.SemaphoreType.DMA,
            pltpu.SemaphoreType.DMA,
            pltpu.SemaphoreType.DMA,
            pltpu.SemaphoreType.DMA,
        ],
    )


_sc_call = _make_kernel()


@jax.jit
def _run(t_idx, c_idx, target_table, context_table, wb):
    return _sc_call(t_idx, c_idx, target_table, context_table, wb)


def kernel(inputs, target_table, context_table, W, b):
    idx = inputs.astype(jnp.int32)
    t_idx = idx[:, 0]
    c_idx = idx[:, 1]
    wb = jnp.concatenate([W.reshape(-1), b,
                          jnp.zeros((L - 1,), jnp.float32)])
    out = _run(t_idx, c_idx, target_table, context_table, wb)
    return out.reshape(BATCH, 1)
